# Initial kernel scaffold; baseline (speedup 1.0000x reference)
import jax
import jax.numpy as jnp
from jax.experimental import pallas as pl


def kernel(pos, area_point, hks, edge_index, weight, face, vertex2face, di_angles, params):
    raise NotImplementedError("write your pallas kernel here")


# speedup vs baseline: 1.2648x; 1.2648x over previous
import functools

import jax
import jax.numpy as jnp
from jax import lax
from jax.experimental import pallas as pl
from jax.experimental.pallas import tpu as pltpu

_INTERPRET = False

N = 10000
E = 160000
BE = 800
BN = 1000


def _silu(x):
    return x * jax.nn.sigmoid(x)


def _dot(a, b):
    return jnp.dot(a, b, preferred_element_type=jnp.float32)


def _pre_body(fi_ref, pos_ref, fW_ref, fb_ref, Wa_ref, Wb_ref, ba_ref,
              ta_ref, tb_ref, x_ref):
    x = _dot(fi_ref[...], fW_ref[...]) + fb_ref[...]
    p = pos_ref[...]
    ta_ref[...] = jnp.concatenate([_dot(x, Wa_ref[...]) + ba_ref[...], p], axis=1)
    tb_ref[...] = jnp.concatenate([_dot(x, Wb_ref[...]), p], axis=1)
    x_ref[...] = x


def _b16(x):
    return x.astype(jnp.bfloat16).astype(jnp.float32)


def _edge_m1(e1, e2, sc, vecs_ref, hid):
    d = e1[:, hid:] - e2[:, hid:]
    radial = (d[:, 0:1] * d[:, 0:1] + d[:, 1:2] * d[:, 1:2]) + d[:, 2:3] * d[:, 2:3]
    m1 = _silu(e1[:, :hid] + e2[:, :hid]
               + _b16(radial) * _b16(vecs_ref[0:1, :])
               + _b16(sc[:, 0:1]) * _b16(vecs_ref[1:2, :])
               + _b16(sc[:, 1:2]) * _b16(vecs_ref[2:3, :]))
    return d, m1


def _edge_body_c(e1_ref, e2_ref, sc_ref, eW2_ref, cW1_ref, cW2_ref, vecs_ref, out_ref, *, hid):
    d, m1 = _edge_m1(e1_ref[...], e2_ref[...], sc_ref[...], vecs_ref, hid)
    m2 = _silu(_dot(m1, eW2_ref[...]) + vecs_ref[3:4, :])
    cm = _silu(_dot(m2, cW1_ref[...]) + vecs_ref[4:5, :])
    c = _dot(cm, cW2_ref[...])
    tr = d * c
    cnt = (lax.broadcasted_iota(jnp.int32, tr.shape, 1) == 3).astype(jnp.float32)
    out_ref[...] = jnp.concatenate([m2, tr + cnt, jnp.zeros_like(tr)], axis=1)


def _edge_body_nc(e1_ref, e2_ref, sc_ref, eW2_ref, vecs_ref, out_ref, *, hid):
    _, m1 = _edge_m1(e1_ref[...], e2_ref[...], sc_ref[...], vecs_ref, hid)
    out_ref[...] = _silu(_dot(m1, eW2_ref[...]) + vecs_ref[3:4, :])


def _node_body(h_ref, acc_ref, pos_ref, nW1_ref, nb1_ref, nW2_ref,
               nb2_ref, Wa_ref, Wb_ref, ba_ref, ta_ref, tb_ref, x_ref, pos_out_ref,
               *, hid):
    acc = acc_ref[...]
    t16 = acc[:, hid:hid + 16]
    cnt = jnp.maximum(t16[:, 3:4], 1.0)
    mask3 = (lax.broadcasted_iota(jnp.int32, t16.shape, 1) < 3).astype(jnp.float32)
    pos_new = pos_ref[...] + t16 * mask3 / cnt
    hm = jnp.concatenate([h_ref[...], acc[:, :hid]], axis=1)
    pre = _silu(_dot(hm, nW1_ref[...]) + nb1_ref[...])
    h_new = _dot(pre, nW2_ref[...]) + nb2_ref[...]
    ta_ref[...] = jnp.concatenate([_dot(h_new, Wa_ref[...]) + ba_ref[...], pos_new], axis=1)
    tb_ref[...] = jnp.concatenate([_dot(h_new, Wb_ref[...]), pos_new], axis=1)
    x_ref[...] = h_new
    pos_out_ref[...] = pos_new


def _node3_body(h_ref, acc_ref, nW1_ref, nb1_ref, nW2_ref, nb2_ref,
                l1W_ref, l1b_ref, l2W_ref, l2b_ref, out_ref):
    hm = jnp.concatenate([h_ref[...], acc_ref[...]], axis=1)
    pre = _silu(_dot(hm, nW1_ref[...]) + nb1_ref[...])
    h_new = _dot(pre, nW2_ref[...]) + nb2_ref[...]
    y = jax.nn.relu(_dot(h_new, l1W_ref[...]) + l1b_ref[...])
    y = _dot(y, l2W_ref[...]) + l2b_ref[...]
    mx = jnp.max(y, axis=1, keepdims=True)
    z = y - mx
    out_ref[...] = z - jnp.log(jnp.sum(jnp.exp(z), axis=1, keepdims=True))


def _full(a):
    return pl.BlockSpec(a.shape, lambda i: (0,) * a.ndim)


def _rows(a, b):
    return pl.BlockSpec((b,) + a.shape[1:], lambda i: (i,) + (0,) * (a.ndim - 1))


def _tc_call(body, grid, ins, in_specs, outs, out_specs):
    return pl.pallas_call(
        body,
        grid=(grid,),
        in_specs=in_specs,
        out_specs=out_specs,
        out_shape=outs,
        interpret=_INTERPRET,
    )(*ins)


def _run_pre(feat_in, pos16, fW, fb, Wa, Wb, ba, hid):
    ins = (feat_in, pos16, fW, fb, Wa, Wb, ba)
    in_specs = [_rows(feat_in, BN), _rows(pos16, BN)] + [_full(a) for a in ins[2:]]
    outs = (jax.ShapeDtypeStruct((N, hid + 16), jnp.float32),
            jax.ShapeDtypeStruct((N, hid + 16), jnp.float32),
            jax.ShapeDtypeStruct((N, 64), jnp.float32))
    out_specs = (pl.BlockSpec((BN, hid + 16), lambda i: (i, 0)),
                 pl.BlockSpec((BN, hid + 16), lambda i: (i, 0)),
                 pl.BlockSpec((BN, 64), lambda i: (i, 0)))
    return _tc_call(_pre_body, N // BN, ins, in_specs, outs, out_specs)


def _run_edge(e1, e2, sc, eW2, cW1, cW2, vecs, hid, with_c):
    wout = hid + 32 if with_c else hid
    if with_c:
        body = functools.partial(_edge_body_c, hid=hid)
        ins = (e1, e2, sc, eW2, cW1, cW2, vecs)
    else:
        body = functools.partial(_edge_body_nc, hid=hid)
        ins = (e1, e2, sc, eW2, vecs)
    in_specs = [_rows(e1, BE), _rows(e2, BE), _rows(sc, BE)] + [_full(a) for a in ins[3:]]
    outs = jax.ShapeDtypeStruct((E, wout), jnp.float32)
    out_specs = pl.BlockSpec((BE, wout), lambda i: (i, 0))
    return _tc_call(body, E // BE, ins, in_specs, outs, out_specs)


def _run_node(h, acc, pos16, nW1, nb1, nW2, nb2, Wa, Wb, ba, hid, fout, hid2):
    ins = (h, acc, pos16, nW1, nb1, nW2, nb2, Wa, Wb, ba)
    in_specs = [_rows(h, BN), _rows(acc, BN), _rows(pos16, BN)] + [_full(a) for a in ins[3:]]
    outs = (jax.ShapeDtypeStruct((N, hid2 + 16), jnp.float32),
            jax.ShapeDtypeStruct((N, hid2 + 16), jnp.float32),
            jax.ShapeDtypeStruct((N, fout), jnp.float32),
            jax.ShapeDtypeStruct((N, 16), jnp.float32))
    out_specs = (pl.BlockSpec((BN, hid2 + 16), lambda i: (i, 0)),
                 pl.BlockSpec((BN, hid2 + 16), lambda i: (i, 0)),
                 pl.BlockSpec((BN, fout), lambda i: (i, 0)),
                 pl.BlockSpec((BN, 16), lambda i: (i, 0)))
    return _tc_call(functools.partial(_node_body, hid=hid), N // BN, ins, in_specs,
                    outs, out_specs)


def _run_node3(h, acc, nW1, nb1, nW2, nb2, l1W, l1b, l2W, l2b):
    ins = (h, acc, nW1, nb1, nW2, nb2, l1W, l1b, l2W, l2b)
    in_specs = [_rows(h, BN), _rows(acc, BN)] + [_full(a) for a in ins[2:]]
    outs = jax.ShapeDtypeStruct((N, 8), jnp.float32)
    out_specs = pl.BlockSpec((BN, 8), lambda i: (i, 0))
    return _tc_call(_node3_body, N // BN, ins, in_specs, outs, out_specs)


def _gather(Ta, Tb, row, col):
    return jnp.take(Ta, row, axis=0), jnp.take(Tb, col, axis=0)


def _scatter_add(data, row):
    return jax.ops.segment_sum(data, row, num_segments=N)


def kernel(pos, area_point, hks, edge_index, weight, face, vertex2face, di_angles, params):
    row, col = edge_index[0], edge_index[1]
    feat_in = jnp.concatenate([area_point[:, None], hks], axis=1)
    pos16 = jnp.pad(pos, ((0, 0), (0, 13)))
    sc = jnp.concatenate([weight, di_angles[:, None]], axis=1)

    lps = params['layers']
    dims = [(64, 128, 64), (128, 256, 128), (256, 512, 256)]

    def split_e(lp, fin):
        Wa = lp['eW1'][:fin]
        Wb = lp['eW1'][fin:2 * fin]
        return Wa, Wb

    def vecs_of(lp, fin, with_c):
        rows_ = [lp['eW1'][2 * fin], lp['eW1'][2 * fin + 1], lp['eW1'][2 * fin + 2],
                 lp['eb2']]
        rows_ += [lp['cb1']] if with_c else [jnp.zeros_like(lp['eb2'])]
        return jnp.stack(rows_)

    fin0, fout0, hid0 = dims[0]
    Wa0, Wb0 = split_e(lps[0], fin0)
    Ta, Tb, x = _run_pre(feat_in, pos16, params['feat_W'], params['feat_b'][None, :],
                         Wa0, Wb0, lps[0]['eb1'][None, :], hid0)

    for li, (fin, fout, hid) in enumerate(dims):
        lp = lps[li]
        with_c = li < 2
        e1, e2 = _gather(Ta, Tb, row, col)
        eout = _run_edge(e1, e2, sc, lp['eW2'], lp['cW1'] if with_c else None,
                         lp['cW2'] if with_c else None,
                         vecs_of(lp, fin, with_c), hid, with_c)
        acc = _scatter_add(eout, row)
        if with_c:
            fin2, fout2, hid2 = dims[li + 1]
            lp2 = lps[li + 1]
            Wa2, Wb2 = split_e(lp2, fin2)
            Ta, Tb, x, pos16 = _run_node(
                x, acc, pos16, lp['nW1'], lp['nb1'][None, :], lp['nW2'],
                lp['nb2'][None, :], Wa2, Wb2, lp2['eb1'][None, :], hid, fout, hid2)
        else:
            out = _run_node3(x, acc, lp['nW1'], lp['nb1'][None, :], lp['nW2'],
                             lp['nb2'][None, :], params['lin1_W'],
                             params['lin1_b'][None, :], params['lin2_W'],
                             params['lin2_b'][None, :])
    return out


# speedup vs baseline: 2.9753x; 2.3524x over previous
import functools

import jax
import jax.numpy as jnp
from jax import lax
from jax.experimental import pallas as pl
from jax.experimental.pallas import tpu as pltpu
from jax.experimental.pallas import tpu_sc as plsc

_INTERPRET = False

N = 10000
E = 160000
BE = 800
BN = 1000


def _silu(x):
    return x * jax.nn.sigmoid(x)


def _dot(a, b):
    return jnp.dot(a, b, preferred_element_type=jnp.float32)


def _pre_body(fi_ref, pos_ref, fW_ref, fb_ref, Wa_ref, Wb_ref, ba_ref,
              ta_ref, tb_ref, x_ref, *, pad):
    x = _dot(fi_ref[...], fW_ref[...]) + fb_ref[...]
    p = pos_ref[...]
    z = jnp.zeros((x.shape[0], pad), jnp.float32)
    ta_ref[...] = jnp.concatenate([_dot(x, Wa_ref[...]) + ba_ref[...], p, z], axis=1)
    tb_ref[...] = jnp.concatenate([_dot(x, Wb_ref[...]), p, z], axis=1)
    x_ref[...] = x


def _b16(x):
    return x.astype(jnp.bfloat16).astype(jnp.float32)


def _edge_m1(e1, e2, sc, vecs_ref, hid):
    d = e1[:, hid:hid + 16] - e2[:, hid:hid + 16]
    radial = (d[:, 0:1] * d[:, 0:1] + d[:, 1:2] * d[:, 1:2]) + d[:, 2:3] * d[:, 2:3]
    m1 = _silu(e1[:, :hid] + e2[:, :hid]
               + _b16(radial) * _b16(vecs_ref[0:1, :])
               + _b16(sc[:, 0:1]) * _b16(vecs_ref[1:2, :])
               + _b16(sc[:, 1:2]) * _b16(vecs_ref[2:3, :]))
    return d, m1


def _edge_body_c(e1_ref, e2_ref, sc_ref, eW2_ref, cW1_ref, cW2_ref, vecs_ref, out_ref,
                 *, hid, pad):
    d, m1 = _edge_m1(e1_ref[...], e2_ref[...], sc_ref[...], vecs_ref, hid)
    m2 = _silu(_dot(m1, eW2_ref[...]) + vecs_ref[3:4, :])
    cm = _silu(_dot(m2, cW1_ref[...]) + vecs_ref[4:5, :])
    c = _dot(cm, cW2_ref[...])
    tr = d * c
    cnt = (lax.broadcasted_iota(jnp.int32, tr.shape, 1) == 3).astype(jnp.float32)
    z = jnp.zeros((tr.shape[0], pad), jnp.float32)
    out_ref[...] = jnp.concatenate([m2, tr + cnt, z], axis=1)


def _edge_body_nc(e1_ref, e2_ref, sc_ref, eW2_ref, vecs_ref, out_ref, *, hid):
    _, m1 = _edge_m1(e1_ref[...], e2_ref[...], sc_ref[...], vecs_ref, hid)
    out_ref[...] = _silu(_dot(m1, eW2_ref[...]) + vecs_ref[3:4, :])


def _node_body(h_ref, acc_ref, *rest, hid, pad, two_acc):
    if two_acc:
        (acc2_ref, pos_ref, nW1_ref, nb1_ref, nW2_ref, nb2_ref, Wa_ref, Wb_ref,
         ba_ref, ta_ref, tb_ref, x_ref, pos_out_ref) = rest
        acc = acc_ref[...] + acc2_ref[...]
    else:
        (pos_ref, nW1_ref, nb1_ref, nW2_ref, nb2_ref, Wa_ref, Wb_ref,
         ba_ref, ta_ref, tb_ref, x_ref, pos_out_ref) = rest
        acc = acc_ref[...]
    t16 = acc[:, hid:hid + 16]
    cnt = jnp.maximum(t16[:, 3:4], 1.0)
    mask3 = (lax.broadcasted_iota(jnp.int32, t16.shape, 1) < 3).astype(jnp.float32)
    pos_new = pos_ref[...] + t16 * mask3 / cnt
    hm = jnp.concatenate([h_ref[...], acc[:, :hid]], axis=1)
    pre = _silu(_dot(hm, nW1_ref[...]) + nb1_ref[...])
    h_new = _dot(pre, nW2_ref[...]) + nb2_ref[...]
    z = jnp.zeros((h_new.shape[0], pad), jnp.float32)
    ta_ref[...] = jnp.concatenate([_dot(h_new, Wa_ref[...]) + ba_ref[...], pos_new, z], axis=1)
    tb_ref[...] = jnp.concatenate([_dot(h_new, Wb_ref[...]), pos_new, z], axis=1)
    x_ref[...] = h_new
    pos_out_ref[...] = pos_new


def _node3_body(h_ref, acc_ref, nW1_ref, nb1_ref, nW2_ref, nb2_ref,
                l1W_ref, l1b_ref, l2W_ref, l2b_ref, out_ref, *, hid):
    hm = jnp.concatenate([h_ref[...], acc_ref[:, :hid]], axis=1)
    pre = _silu(_dot(hm, nW1_ref[...]) + nb1_ref[...])
    h_new = _dot(pre, nW2_ref[...]) + nb2_ref[...]
    y = jax.nn.relu(_dot(h_new, l1W_ref[...]) + l1b_ref[...])
    y = _dot(y, l2W_ref[...]) + l2b_ref[...]
    mx = jnp.max(y, axis=1, keepdims=True)
    z = y - mx
    out_ref[...] = z - jnp.log(jnp.sum(jnp.exp(z), axis=1, keepdims=True))


def _full(a):
    return pl.BlockSpec(a.shape, lambda i: (0,) * a.ndim)


def _rows(a, b):
    return pl.BlockSpec((b,) + a.shape[1:], lambda i: (i,) + (0,) * (a.ndim - 1))


def _tc_call(body, grid, ins, in_specs, outs, out_specs):
    return pl.pallas_call(
        body,
        grid=(grid,),
        in_specs=in_specs,
        out_specs=out_specs,
        out_shape=outs,
        interpret=_INTERPRET,
    )(*ins)


def _tpad(hid):
    return -(-(hid + 16) // 128) * 128


def _run_pre(feat_in, pos16, fW, fb, Wa, Wb, ba, hid):
    wt = _tpad(hid)
    ins = (feat_in, pos16, fW, fb, Wa, Wb, ba)
    in_specs = [_rows(feat_in, BN), _rows(pos16, BN)] + [_full(a) for a in ins[2:]]
    outs = (jax.ShapeDtypeStruct((N, wt), jnp.float32),
            jax.ShapeDtypeStruct((N, wt), jnp.float32),
            jax.ShapeDtypeStruct((N, 64), jnp.float32))
    out_specs = (pl.BlockSpec((BN, wt), lambda i: (i, 0)),
                 pl.BlockSpec((BN, wt), lambda i: (i, 0)),
                 pl.BlockSpec((BN, 64), lambda i: (i, 0)))
    return _tc_call(functools.partial(_pre_body, pad=wt - hid - 16),
                    N // BN, ins, in_specs, outs, out_specs)


def _run_edge(e1, e2, sc, eW2, cW1, cW2, vecs, hid, with_c):
    wout = -(-(hid + 16) // 128) * 128 if with_c else hid
    if with_c:
        body = functools.partial(_edge_body_c, hid=hid, pad=wout - hid - 16)
        ins = (e1, e2, sc, eW2, cW1, cW2, vecs)
    else:
        body = functools.partial(_edge_body_nc, hid=hid)
        ins = (e1, e2, sc, eW2, vecs)
    in_specs = [_rows(e1, BE), _rows(e2, BE), _rows(sc, BE)] + [_full(a) for a in ins[3:]]
    outs = jax.ShapeDtypeStruct((E, wout), jnp.float32)
    out_specs = pl.BlockSpec((BE, wout), lambda i: (i, 0))
    return _tc_call(body, E // BE, ins, in_specs, outs, out_specs)


def _run_node(h, accs, pos16, nW1, nb1, nW2, nb2, Wa, Wb, ba, hid, fout, hid2):
    wt = _tpad(hid2)
    two_acc = len(accs) == 2
    ins = (h,) + tuple(accs) + (pos16, nW1, nb1, nW2, nb2, Wa, Wb, ba)
    in_specs = ([_rows(h, BN)] + [_rows(a, BN) for a in accs] + [_rows(pos16, BN)]
                + [_full(a) for a in ins[2 + len(accs):]])
    outs = (jax.ShapeDtypeStruct((N, wt), jnp.float32),
            jax.ShapeDtypeStruct((N, wt), jnp.float32),
            jax.ShapeDtypeStruct((N, fout), jnp.float32),
            jax.ShapeDtypeStruct((N, 16), jnp.float32))
    out_specs = (pl.BlockSpec((BN, wt), lambda i: (i, 0)),
                 pl.BlockSpec((BN, wt), lambda i: (i, 0)),
                 pl.BlockSpec((BN, fout), lambda i: (i, 0)),
                 pl.BlockSpec((BN, 16), lambda i: (i, 0)))
    return _tc_call(functools.partial(_node_body, hid=hid, pad=wt - hid2 - 16,
                                      two_acc=two_acc),
                    N // BN, ins, in_specs, outs, out_specs)


def _run_node3(h, acc, nW1, nb1, nW2, nb2, l1W, l1b, l2W, l2b, hid):
    ins = (h, acc, nW1, nb1, nW2, nb2, l1W, l1b, l2W, l2b)
    in_specs = [_rows(h, BN), _rows(acc, BN)] + [_full(a) for a in ins[2:]]
    outs = jax.ShapeDtypeStruct((N, 8), jnp.float32)
    out_specs = pl.BlockSpec((BN, 8), lambda i: (i, 0))
    return _tc_call(functools.partial(_node3_body, hid=hid), N // BN, ins, in_specs,
                    outs, out_specs)


_NW = 32
_EB = 128
_NBLK = E // _EB


def _gather(Ta, Tb, row, col):
    Wg = Ta.shape[1]
    mesh = plsc.VectorSubcoreMesh(core_axis_name="c", subcore_axis_name="s")

    @functools.partial(
        pl.kernel, mesh=mesh,
        out_type=(jax.ShapeDtypeStruct((E, Wg), jnp.float32),
                  jax.ShapeDtypeStruct((E, Wg), jnp.float32)),
        scratch_types=[pltpu.VMEM((_EB,), jnp.int32),
                       pltpu.VMEM((_EB,), jnp.int32),
                       pltpu.VMEM((_EB, Wg), jnp.float32),
                       pltpu.VMEM((_EB, Wg), jnp.float32),
                       pltpu.SemaphoreType.DMA,
                       pltpu.SemaphoreType.DMA],
    )
    def k(ta_hbm, tb_hbm, row_hbm, col_hbm, e1_hbm, e2_hbm,
          idx1_v, idx2_v, rows1_v, rows2_v, sem1, sem2):
        wid = lax.axis_index("s") * 2 + lax.axis_index("c")

        def body(kk, carry):
            b = kk * _NW + wid

            @pl.when(b < _NBLK)
            def _():
                base = b * _EB
                pltpu.sync_copy(row_hbm.at[pl.ds(base, _EB)], idx1_v)
                pltpu.sync_copy(col_hbm.at[pl.ds(base, _EB)], idx2_v)
                cp1 = pltpu.async_copy(ta_hbm.at[idx1_v], rows1_v, sem1)
                cp2 = pltpu.async_copy(tb_hbm.at[idx2_v], rows2_v, sem2)
                cp1.wait()
                cp2.wait()
                pltpu.sync_copy(rows1_v, e1_hbm.at[pl.ds(base, _EB)])
                pltpu.sync_copy(rows2_v, e2_hbm.at[pl.ds(base, _EB)])
            return carry

        lax.fori_loop(0, (_NBLK + _NW - 1) // _NW, body, 0)

    return k(Ta, Tb, row, col)


def _scatter_add(data, row, split_cols):
    Ws = data.shape[1]
    zeros = jnp.zeros((640, 128), jnp.float32)
    mesh = plsc.VectorSubcoreMesh(core_axis_name="c", subcore_axis_name="s")
    if split_cols:
        out_t = jax.ShapeDtypeStruct((N, 256), jnp.float32)
    else:
        out_t = jax.ShapeDtypeStruct((2, N, 128), jnp.float32)

    @functools.partial(
        pl.kernel, mesh=mesh,
        out_type=out_t,
        scratch_types=[pltpu.VMEM_SHARED((N, 128), jnp.float32),
                       pltpu.VMEM((_EB,), jnp.int32),
                       pltpu.VMEM((_EB, 128), jnp.float32)],
    )
    def k(data_hbm, row_hbm, z_hbm, out_hbm, acc_sh, idx_v, buf_v):
        cid = lax.axis_index("c")
        sid = lax.axis_index("s")

        def over_rows(fn):
            @pl.when(sid < 15)
            def _():
                fn(pl.multiple_of(sid * 640, 8), 640)

            @pl.when(sid == 15)
            def _():
                fn(9600, 400)

        over_rows(lambda off, sz: pltpu.sync_copy(z_hbm.at[pl.ds(0, sz)],
                                                  acc_sh.at[pl.ds(off, sz)]))
        plsc.subcore_barrier()

        if split_cols:
            def body(kk, carry):
                b = kk * 16 + sid

                @pl.when(b < _NBLK)
                def _():
                    base = b * _EB
                    pltpu.sync_copy(row_hbm.at[pl.ds(base, _EB)], idx_v)
                    pltpu.sync_copy(
                        data_hbm.at[pl.ds(base, _EB),
                                    pl.ds(pl.multiple_of(cid * 128, 128), 128)],
                        buf_v)
                    pltpu.sync_copy(buf_v, acc_sh.at[idx_v], add=True)
                return carry

            lax.fori_loop(0, (_NBLK + 15) // 16, body, 0)
            plsc.subcore_barrier()
            over_rows(lambda off, sz: pltpu.sync_copy(
                acc_sh.at[pl.ds(off, sz)],
                out_hbm.at[pl.ds(off, sz),
                           pl.ds(pl.multiple_of(cid * 128, 128), 128)]))
        else:
            def body(kk, carry):
                b = kk * _NW + sid * 2 + cid

                @pl.when(b < _NBLK)
                def _():
                    base = b * _EB
                    pltpu.sync_copy(row_hbm.at[pl.ds(base, _EB)], idx_v)
                    pltpu.sync_copy(data_hbm.at[pl.ds(base, _EB)], buf_v)
                    pltpu.sync_copy(buf_v, acc_sh.at[idx_v], add=True)
                return carry

            lax.fori_loop(0, (_NBLK + _NW - 1) // _NW, body, 0)
            plsc.subcore_barrier()
            over_rows(lambda off, sz: pltpu.sync_copy(
                acc_sh.at[pl.ds(off, sz)], out_hbm.at[cid, pl.ds(off, sz)]))

    return k(data, row, zeros)


def kernel(pos, area_point, hks, edge_index, weight, face, vertex2face, di_angles, params):
    row, col = edge_index[0], edge_index[1]
    feat_in = jnp.concatenate([area_point[:, None], hks], axis=1)
    pos16 = jnp.pad(pos, ((0, 0), (0, 13)))
    sc = jnp.concatenate([weight, di_angles[:, None]], axis=1)

    lps = params['layers']
    dims = [(64, 128, 64), (128, 256, 128), (256, 512, 256)]

    def split_e(lp, fin):
        Wa = lp['eW1'][:fin]
        Wb = lp['eW1'][fin:2 * fin]
        return Wa, Wb

    def vecs_of(lp, fin, with_c):
        rows_ = [lp['eW1'][2 * fin], lp['eW1'][2 * fin + 1], lp['eW1'][2 * fin + 2],
                 lp['eb2']]
        rows_ += [lp['cb1']] if with_c else [jnp.zeros_like(lp['eb2'])]
        return jnp.stack(rows_)

    fin0, fout0, hid0 = dims[0]
    Wa0, Wb0 = split_e(lps[0], fin0)
    Ta, Tb, x = _run_pre(feat_in, pos16, params['feat_W'], params['feat_b'][None, :],
                         Wa0, Wb0, lps[0]['eb1'][None, :], hid0)

    for li, (fin, fout, hid) in enumerate(dims):
        lp = lps[li]
        with_c = li < 2
        e1, e2 = _gather(Ta, Tb, row, col)
        eout = _run_edge(e1, e2, sc, lp['eW2'], lp['cW1'] if with_c else None,
                         lp['cW2'] if with_c else None,
                         vecs_of(lp, fin, with_c), hid, with_c)
        acc = _scatter_add(eout, row, split_cols=eout.shape[1] > 128)
        accs = (acc,) if acc.ndim == 2 else (acc[0], acc[1])
        if with_c:
            fin2, fout2, hid2 = dims[li + 1]
            lp2 = lps[li + 1]
            Wa2, Wb2 = split_e(lp2, fin2)
            Ta, Tb, x, pos16 = _run_node(
                x, accs, pos16, lp['nW1'], lp['nb1'][None, :], lp['nW2'],
                lp['nb2'][None, :], Wa2, Wb2, lp2['eb1'][None, :], hid, fout, hid2)
        else:
            out = _run_node3(x, accs[0], lp['nW1'], lp['nb1'][None, :], lp['nW2'],
                             lp['nb2'][None, :], params['lin1_W'],
                             params['lin1_b'][None, :], params['lin2_W'],
                             params['lin2_b'][None, :], hid)
    return out


# speedup vs baseline: 3.5351x; 1.1882x over previous
import functools

import jax
import jax.numpy as jnp
from jax import lax
from jax.experimental import pallas as pl
from jax.experimental.pallas import tpu as pltpu
from jax.experimental.pallas import tpu_sc as plsc

_INTERPRET = False

N = 10000
E = 160000
BE = 800
BN = 1000


def _silu(x):
    return x * jax.nn.sigmoid(x)


def _dot(a, b):
    return jnp.dot(a, b, preferred_element_type=jnp.float32)


def _pre_body(fi_ref, pos_ref, fW_ref, fb_ref, Wa_ref, Wb_ref, ba_ref,
              ta_ref, tb_ref, x_ref, *, pad):
    x = _dot(fi_ref[...], fW_ref[...]) + fb_ref[...]
    p = pos_ref[...]
    z = jnp.zeros((x.shape[0], pad), jnp.float32)
    ta_ref[...] = jnp.concatenate([_dot(x, Wa_ref[...]) + ba_ref[...], p, z], axis=1)
    tb_ref[...] = jnp.concatenate([_dot(x, Wb_ref[...]), -p, z], axis=1)
    x_ref[...] = x


def _b16(x):
    return x.astype(jnp.bfloat16).astype(jnp.float32)


def _edge_m1(e1, e2, sc, vecs_ref, hid):
    d = e1[:, hid:hid + 16] + e2[:, hid:hid + 16]
    radial = (d[:, 0:1] * d[:, 0:1] + d[:, 1:2] * d[:, 1:2]) + d[:, 2:3] * d[:, 2:3]
    m1 = _silu(e1[:, :hid] + e2[:, :hid]
               + _b16(radial) * _b16(vecs_ref[0:1, :])
               + _b16(sc[:, 0:1]) * _b16(vecs_ref[1:2, :])
               + _b16(sc[:, 1:2]) * _b16(vecs_ref[2:3, :]))
    return d, m1


def _edge_body_c(e1_ref, e2_ref, sc_ref, eW2_ref, cW1_ref, cW2_ref, vecs_ref, out_ref,
                 *, hid, pad):
    d, m1 = _edge_m1(e1_ref[...], e2_ref[...], sc_ref[...], vecs_ref, hid)
    m2 = _silu(_dot(m1, eW2_ref[...]) + vecs_ref[3:4, :])
    cm = _silu(_dot(m2, cW1_ref[...]) + vecs_ref[4:5, :])
    c = _dot(cm, cW2_ref[...])
    tr = d * c
    cnt = (lax.broadcasted_iota(jnp.int32, tr.shape, 1) == 3).astype(jnp.float32)
    z = jnp.zeros((tr.shape[0], pad), jnp.float32)
    out_ref[...] = jnp.concatenate([m2, tr + cnt, z], axis=1)


def _edge_body_nc(e1_ref, e2_ref, sc_ref, eW2_ref, vecs_ref, out_ref, *, hid):
    _, m1 = _edge_m1(e1_ref[...], e2_ref[...], sc_ref[...], vecs_ref, hid)
    out_ref[...] = _silu(_dot(m1, eW2_ref[...]) + vecs_ref[3:4, :])


def _node_body(h_ref, *rest, hid, pad, n_acc):
    acc = rest[0][...]
    for r in rest[1:n_acc]:
        acc = acc + r[...]
    (pos_ref, nW1_ref, nb1_ref, nW2_ref, nb2_ref, Wa_ref, Wb_ref,
     ba_ref, ta_ref, tb_ref, x_ref, pos_out_ref) = rest[n_acc:]
    t16 = acc[:, hid:hid + 16]
    cnt = jnp.maximum(t16[:, 3:4], 1.0)
    mask3 = (lax.broadcasted_iota(jnp.int32, t16.shape, 1) < 3).astype(jnp.float32)
    pos_new = pos_ref[...] + t16 * mask3 / cnt
    hm = jnp.concatenate([h_ref[...], acc[:, :hid]], axis=1)
    pre = _silu(_dot(hm, nW1_ref[...]) + nb1_ref[...])
    h_new = _dot(pre, nW2_ref[...]) + nb2_ref[...]
    z = jnp.zeros((h_new.shape[0], pad), jnp.float32)
    ta_ref[...] = jnp.concatenate([_dot(h_new, Wa_ref[...]) + ba_ref[...], pos_new, z], axis=1)
    tb_ref[...] = jnp.concatenate([_dot(h_new, Wb_ref[...]), -pos_new, z], axis=1)
    x_ref[...] = h_new
    pos_out_ref[...] = pos_new


def _node3_body(h_ref, *rest, hid, n_acc):
    acc = rest[0][...]
    for r in rest[1:n_acc]:
        acc = acc + r[...]
    (nW1_ref, nb1_ref, nW2_ref, nb2_ref,
     l1W_ref, l1b_ref, l2W_ref, l2b_ref, out_ref) = rest[n_acc:]
    hm = jnp.concatenate([h_ref[...], acc[:, :hid]], axis=1)
    pre = _silu(_dot(hm, nW1_ref[...]) + nb1_ref[...])
    h_new = _dot(pre, nW2_ref[...]) + nb2_ref[...]
    y = jax.nn.relu(_dot(h_new, l1W_ref[...]) + l1b_ref[...])
    y = _dot(y, l2W_ref[...]) + l2b_ref[...]
    mx = jnp.max(y, axis=1, keepdims=True)
    z = y - mx
    out_ref[...] = z - jnp.log(jnp.sum(jnp.exp(z), axis=1, keepdims=True))


def _full(a):
    return pl.BlockSpec(a.shape, lambda i: (0,) * a.ndim)


def _rows(a, b):
    return pl.BlockSpec((b,) + a.shape[1:], lambda i: (i,) + (0,) * (a.ndim - 1))


def _tc_call(body, grid, ins, in_specs, outs, out_specs):
    return pl.pallas_call(
        body,
        grid=(grid,),
        in_specs=in_specs,
        out_specs=out_specs,
        out_shape=outs,
        interpret=_INTERPRET,
    )(*ins)


def _tpad(hid):
    return -(-(hid + 16) // 128) * 128


def _run_pre(feat_in, pos16, fW, fb, Wa, Wb, ba, hid):
    wt = _tpad(hid)
    ins = (feat_in, pos16, fW, fb, Wa, Wb, ba)
    in_specs = [_rows(feat_in, BN), _rows(pos16, BN)] + [_full(a) for a in ins[2:]]
    outs = (jax.ShapeDtypeStruct((N, wt), jnp.float32),
            jax.ShapeDtypeStruct((N, wt), jnp.float32),
            jax.ShapeDtypeStruct((N, 64), jnp.float32))
    out_specs = (pl.BlockSpec((BN, wt), lambda i: (i, 0)),
                 pl.BlockSpec((BN, wt), lambda i: (i, 0)),
                 pl.BlockSpec((BN, 64), lambda i: (i, 0)))
    return _tc_call(functools.partial(_pre_body, pad=wt - hid - 16),
                    N // BN, ins, in_specs, outs, out_specs)


def _run_edge(e1, e2, sc, eW2, cW1, cW2, vecs, hid, with_c):
    ne = e1.shape[0]
    wout = -(-(hid + 16) // 128) * 128 if with_c else hid
    if with_c:
        body = functools.partial(_edge_body_c, hid=hid, pad=wout - hid - 16)
        ins = (e1, e2, sc, eW2, cW1, cW2, vecs)
    else:
        body = functools.partial(_edge_body_nc, hid=hid)
        ins = (e1, e2, sc, eW2, vecs)
    in_specs = [_rows(e1, BE), _rows(e2, BE), _rows(sc, BE)] + [_full(a) for a in ins[3:]]
    outs = jax.ShapeDtypeStruct((ne, wout), jnp.float32)
    out_specs = pl.BlockSpec((BE, wout), lambda i: (i, 0))
    return _tc_call(body, ne // BE, ins, in_specs, outs, out_specs)


def _run_node(h, accs, pos16, nW1, nb1, nW2, nb2, Wa, Wb, ba, hid, fout, hid2):
    wt = _tpad(hid2)
    n_acc = len(accs)
    ins = (h,) + tuple(accs) + (pos16, nW1, nb1, nW2, nb2, Wa, Wb, ba)
    in_specs = ([_rows(h, BN)] + [_rows(a, BN) for a in accs] + [_rows(pos16, BN)]
                + [_full(a) for a in ins[2 + n_acc:]])
    outs = (jax.ShapeDtypeStruct((N, wt), jnp.float32),
            jax.ShapeDtypeStruct((N, wt), jnp.float32),
            jax.ShapeDtypeStruct((N, fout), jnp.float32),
            jax.ShapeDtypeStruct((N, 16), jnp.float32))
    out_specs = (pl.BlockSpec((BN, wt), lambda i: (i, 0)),
                 pl.BlockSpec((BN, wt), lambda i: (i, 0)),
                 pl.BlockSpec((BN, fout), lambda i: (i, 0)),
                 pl.BlockSpec((BN, 16), lambda i: (i, 0)))
    return _tc_call(functools.partial(_node_body, hid=hid, pad=wt - hid2 - 16,
                                      n_acc=n_acc),
                    N // BN, ins, in_specs, outs, out_specs)


def _run_node3(h, accs, nW1, nb1, nW2, nb2, l1W, l1b, l2W, l2b, hid):
    ins = (h,) + tuple(accs) + (nW1, nb1, nW2, nb2, l1W, l1b, l2W, l2b)
    in_specs = ([_rows(h, BN)] + [_rows(a, BN) for a in accs]
                + [_full(a) for a in ins[1 + len(accs):]])
    outs = jax.ShapeDtypeStruct((N, 8), jnp.float32)
    out_specs = pl.BlockSpec((BN, 8), lambda i: (i, 0))
    return _tc_call(functools.partial(_node3_body, hid=hid, n_acc=len(accs)),
                    N // BN, ins, in_specs, outs, out_specs)


_NW = 32
_EB = 128
_NBLK = E // _EB


def _gather(Ta, Tb, row, col):
    Wg = Ta.shape[1]
    ne = row.shape[0]
    nblk = ne // _EB
    mesh = plsc.VectorSubcoreMesh(core_axis_name="c", subcore_axis_name="s")

    @functools.partial(
        pl.kernel, mesh=mesh,
        out_type=(jax.ShapeDtypeStruct((ne, Wg), jnp.float32),
                  jax.ShapeDtypeStruct((ne, Wg), jnp.float32)),
        scratch_types=[pltpu.VMEM((_EB,), jnp.int32),
                       pltpu.VMEM((_EB,), jnp.int32),
                       pltpu.VMEM((_EB, Wg), jnp.float32),
                       pltpu.VMEM((_EB, Wg), jnp.float32),
                       pltpu.SemaphoreType.DMA,
                       pltpu.SemaphoreType.DMA],
    )
    def k(ta_hbm, tb_hbm, row_hbm, col_hbm, e1_hbm, e2_hbm,
          idx1_v, idx2_v, rows1_v, rows2_v, sem1, sem2):
        wid = lax.axis_index("s") * 2 + lax.axis_index("c")

        def body(kk, carry):
            b = kk * _NW + wid

            @pl.when(b < nblk)
            def _():
                base = b * _EB
                pltpu.sync_copy(row_hbm.at[pl.ds(base, _EB)], idx1_v)
                pltpu.sync_copy(col_hbm.at[pl.ds(base, _EB)], idx2_v)
                cp1 = pltpu.async_copy(ta_hbm.at[idx1_v], rows1_v, sem1)
                cp2 = pltpu.async_copy(tb_hbm.at[idx2_v], rows2_v, sem2)
                cp1.wait()
                cp2.wait()
                pltpu.sync_copy(rows1_v, e1_hbm.at[pl.ds(base, _EB)])
                pltpu.sync_copy(rows2_v, e2_hbm.at[pl.ds(base, _EB)])
            return carry

        lax.fori_loop(0, (nblk + _NW - 1) // _NW, body, 0)

    return k(Ta, Tb, row, col)


def _scatter_add(data, row, split_cols):
    Ws = data.shape[1]
    nblk = data.shape[0] // _EB
    zeros = jnp.zeros((640, 128), jnp.float32)
    mesh = plsc.VectorSubcoreMesh(core_axis_name="c", subcore_axis_name="s")
    if split_cols:
        out_t = jax.ShapeDtypeStruct((N, 256), jnp.float32)
    else:
        out_t = jax.ShapeDtypeStruct((2, N, 128), jnp.float32)

    @functools.partial(
        pl.kernel, mesh=mesh,
        out_type=out_t,
        scratch_types=[pltpu.VMEM_SHARED((N, 128), jnp.float32),
                       pltpu.VMEM((_EB,), jnp.int32),
                       pltpu.VMEM((_EB, 128), jnp.float32)],
    )
    def k(data_hbm, row_hbm, z_hbm, out_hbm, acc_sh, idx_v, buf_v):
        cid = lax.axis_index("c")
        sid = lax.axis_index("s")

        def over_rows(fn):
            @pl.when(sid < 15)
            def _():
                fn(pl.multiple_of(sid * 640, 8), 640)

            @pl.when(sid == 15)
            def _():
                fn(9600, 400)

        over_rows(lambda off, sz: pltpu.sync_copy(z_hbm.at[pl.ds(0, sz)],
                                                  acc_sh.at[pl.ds(off, sz)]))
        plsc.subcore_barrier()

        if split_cols:
            def body(kk, carry):
                b = kk * 16 + sid

                @pl.when(b < nblk)
                def _():
                    base = b * _EB
                    pltpu.sync_copy(row_hbm.at[pl.ds(base, _EB)], idx_v)
                    pltpu.sync_copy(
                        data_hbm.at[pl.ds(base, _EB),
                                    pl.ds(pl.multiple_of(cid * 128, 128), 128)],
                        buf_v)
                    pltpu.sync_copy(buf_v, acc_sh.at[idx_v], add=True)
                return carry

            lax.fori_loop(0, (nblk + 15) // 16, body, 0)
            plsc.subcore_barrier()
            over_rows(lambda off, sz: pltpu.sync_copy(
                acc_sh.at[pl.ds(off, sz)],
                out_hbm.at[pl.ds(off, sz),
                           pl.ds(pl.multiple_of(cid * 128, 128), 128)]))
        else:
            def body(kk, carry):
                b = kk * _NW + sid * 2 + cid

                @pl.when(b < nblk)
                def _():
                    base = b * _EB
                    pltpu.sync_copy(row_hbm.at[pl.ds(base, _EB)], idx_v)
                    pltpu.sync_copy(data_hbm.at[pl.ds(base, _EB)], buf_v)
                    pltpu.sync_copy(buf_v, acc_sh.at[idx_v], add=True)
                return carry

            lax.fori_loop(0, (nblk + _NW - 1) // _NW, body, 0)
            plsc.subcore_barrier()
            over_rows(lambda off, sz: pltpu.sync_copy(
                acc_sh.at[pl.ds(off, sz)], out_hbm.at[cid, pl.ds(off, sz)]))

    return k(data, row, zeros)


def kernel(pos, area_point, hks, edge_index, weight, face, vertex2face, di_angles, params):
    row, col = edge_index[0], edge_index[1]
    feat_in = jnp.concatenate([area_point[:, None], hks], axis=1)
    pos16 = jnp.pad(pos, ((0, 0), (0, 13)))
    sc = jnp.concatenate([weight, di_angles[:, None]], axis=1)

    lps = params['layers']
    dims = [(64, 128, 64), (128, 256, 128), (256, 512, 256)]

    def split_e(lp, fin):
        Wa = lp['eW1'][:fin]
        Wb = lp['eW1'][fin:2 * fin]
        return Wa, Wb

    def vecs_of(lp, fin, with_c):
        rows_ = [lp['eW1'][2 * fin], lp['eW1'][2 * fin + 1], lp['eW1'][2 * fin + 2],
                 lp['eb2']]
        rows_ += [lp['cb1']] if with_c else [jnp.zeros_like(lp['eb2'])]
        return jnp.stack(rows_)

    fin0, fout0, hid0 = dims[0]
    Wa0, Wb0 = split_e(lps[0], fin0)
    Ta, Tb, x = _run_pre(feat_in, pos16, params['feat_W'], params['feat_b'][None, :],
                         Wa0, Wb0, lps[0]['eb1'][None, :], hid0)

    parts = 2
    ep = E // parts

    for li, (fin, fout, hid) in enumerate(dims):
        lp = lps[li]
        with_c = li < 2
        accs = []
        for p in range(parts):
            sl = slice(p * ep, (p + 1) * ep)
            e1, e2 = _gather(Ta, Tb, row[sl], col[sl])
            eout = _run_edge(e1, e2, sc[sl], lp['eW2'],
                             lp['cW1'] if with_c else None,
                             lp['cW2'] if with_c else None,
                             vecs_of(lp, fin, with_c), hid, with_c)
            acc = _scatter_add(eout, row[sl], split_cols=eout.shape[1] > 128)
            accs += [acc] if acc.ndim == 2 else [acc[0], acc[1]]
        if with_c:
            fin2, fout2, hid2 = dims[li + 1]
            lp2 = lps[li + 1]
            Wa2, Wb2 = split_e(lp2, fin2)
            Ta, Tb, x, pos16 = _run_node(
                x, accs, pos16, lp['nW1'], lp['nb1'][None, :], lp['nW2'],
                lp['nb2'][None, :], Wa2, Wb2, lp2['eb1'][None, :], hid, fout, hid2)
        else:
            out = _run_node3(x, accs, lp['nW1'], lp['nb1'][None, :], lp['nW2'],
                             lp['nb2'][None, :], params['lin1_W'],
                             params['lin1_b'][None, :], params['lin2_W'],
                             params['lin2_b'][None, :], hid)
    return out


# speedup vs baseline: 3.5361x; 1.0003x over previous
import functools

import jax
import jax.numpy as jnp
from jax import lax
from jax.experimental import pallas as pl
from jax.experimental.pallas import tpu as pltpu
from jax.experimental.pallas import tpu_sc as plsc

N = 10000
E = 160000
BE = 800
BN = 1000


def _silu(x):
    return x * jax.nn.sigmoid(x)


def _dot(a, b):
    return jnp.dot(a, b, preferred_element_type=jnp.float32)


def _pre_body(fi_ref, pos_ref, fW_ref, fb_ref, Wa_ref, Wb_ref, ba_ref,
              ta_ref, tb_ref, x_ref, *, pad):
    x = _dot(fi_ref[...], fW_ref[...]) + fb_ref[...]
    p = pos_ref[...]
    z = jnp.zeros((x.shape[0], pad), jnp.float32)
    ta_ref[...] = jnp.concatenate([_dot(x, Wa_ref[...]) + ba_ref[...], p, z], axis=1)
    tb_ref[...] = jnp.concatenate([_dot(x, Wb_ref[...]), -p, z], axis=1)
    x_ref[...] = x


def _b16(x):
    return x.astype(jnp.bfloat16).astype(jnp.float32)


def _edge_m1(e1, e2, sc, vecs_ref, hid):
    d = e1[:, hid:hid + 16] + e2[:, hid:hid + 16]
    radial = (d[:, 0:1] * d[:, 0:1] + d[:, 1:2] * d[:, 1:2]) + d[:, 2:3] * d[:, 2:3]
    m1 = _silu(e1[:, :hid] + e2[:, :hid]
               + _b16(radial) * _b16(vecs_ref[0:1, :])
               + _b16(sc[:, 0:1]) * _b16(vecs_ref[1:2, :])
               + _b16(sc[:, 1:2]) * _b16(vecs_ref[2:3, :]))
    return d, m1


def _edge_body_c(e1_ref, e2_ref, sc_ref, eW2_ref, cW1_ref, cW2_ref, vecs_ref, out_ref,
                 *, hid, pad):
    d, m1 = _edge_m1(e1_ref[...], e2_ref[...], sc_ref[...], vecs_ref, hid)
    m2 = _silu(_dot(m1, eW2_ref[...]) + vecs_ref[3:4, :])
    cm = _silu(_dot(m2, cW1_ref[...]) + vecs_ref[4:5, :])
    c = _dot(cm, cW2_ref[...])
    tr = d * c
    cnt = (lax.broadcasted_iota(jnp.int32, tr.shape, 1) == 3).astype(jnp.float32)
    z = jnp.zeros((tr.shape[0], pad), jnp.float32)
    out_ref[...] = jnp.concatenate([m2, tr + cnt, z], axis=1)


def _edge_body_nc(e1_ref, e2_ref, sc_ref, eW2_ref, vecs_ref, out_ref, *, hid):
    _, m1 = _edge_m1(e1_ref[...], e2_ref[...], sc_ref[...], vecs_ref, hid)
    out_ref[...] = _silu(_dot(m1, eW2_ref[...]) + vecs_ref[3:4, :])


def _node_body(h_ref, *rest, hid, pad, n_acc):
    acc = rest[0][...]
    for r in rest[1:n_acc]:
        acc = acc + r[...]
    (pos_ref, nW1_ref, nb1_ref, nW2_ref, nb2_ref, Wa_ref, Wb_ref,
     ba_ref, ta_ref, tb_ref, x_ref, pos_out_ref) = rest[n_acc:]
    t16 = acc[:, hid:hid + 16]
    cnt = jnp.maximum(t16[:, 3:4], 1.0)
    mask3 = (lax.broadcasted_iota(jnp.int32, t16.shape, 1) < 3).astype(jnp.float32)
    pos_new = pos_ref[...] + t16 * mask3 / cnt
    hm = jnp.concatenate([h_ref[...], acc[:, :hid]], axis=1)
    pre = _silu(_dot(hm, nW1_ref[...]) + nb1_ref[...])
    h_new = _dot(pre, nW2_ref[...]) + nb2_ref[...]
    z = jnp.zeros((h_new.shape[0], pad), jnp.float32)
    ta_ref[...] = jnp.concatenate([_dot(h_new, Wa_ref[...]) + ba_ref[...], pos_new, z], axis=1)
    tb_ref[...] = jnp.concatenate([_dot(h_new, Wb_ref[...]), -pos_new, z], axis=1)
    x_ref[...] = h_new
    pos_out_ref[...] = pos_new


def _node3_body(h_ref, *rest, hid, n_acc):
    acc = rest[0][...]
    for r in rest[1:n_acc]:
        acc = acc + r[...]
    (nW1_ref, nb1_ref, nW2_ref, nb2_ref,
     l1W_ref, l1b_ref, l2W_ref, l2b_ref, out_ref) = rest[n_acc:]
    hm = jnp.concatenate([h_ref[...], acc[:, :hid]], axis=1)
    pre = _silu(_dot(hm, nW1_ref[...]) + nb1_ref[...])
    h_new = _dot(pre, nW2_ref[...]) + nb2_ref[...]
    y = jax.nn.relu(_dot(h_new, l1W_ref[...]) + l1b_ref[...])
    y = _dot(y, l2W_ref[...]) + l2b_ref[...]
    mx = jnp.max(y, axis=1, keepdims=True)
    z = y - mx
    out_ref[...] = z - jnp.log(jnp.sum(jnp.exp(z), axis=1, keepdims=True))


def _full(a):
    return pl.BlockSpec(a.shape, lambda i: (0,) * a.ndim)


def _rows(a, b):
    return pl.BlockSpec((b,) + a.shape[1:], lambda i: (i,) + (0,) * (a.ndim - 1))


def _tc_call(body, grid, ins, in_specs, outs, out_specs):
    return pl.pallas_call(
        body,
        grid=(grid,),
        in_specs=in_specs,
        out_specs=out_specs,
        out_shape=outs,
    )(*ins)


def _tpad(hid):
    return -(-(hid + 16) // 128) * 128


def _run_pre(feat_in, pos16, fW, fb, Wa, Wb, ba, hid):
    wt = _tpad(hid)
    ins = (feat_in, pos16, fW, fb, Wa, Wb, ba)
    in_specs = [_rows(feat_in, BN), _rows(pos16, BN)] + [_full(a) for a in ins[2:]]
    outs = (jax.ShapeDtypeStruct((N, wt), jnp.float32),
            jax.ShapeDtypeStruct((N, wt), jnp.float32),
            jax.ShapeDtypeStruct((N, 64), jnp.float32))
    out_specs = (pl.BlockSpec((BN, wt), lambda i: (i, 0)),
                 pl.BlockSpec((BN, wt), lambda i: (i, 0)),
                 pl.BlockSpec((BN, 64), lambda i: (i, 0)))
    return _tc_call(functools.partial(_pre_body, pad=wt - hid - 16),
                    N // BN, ins, in_specs, outs, out_specs)


def _run_edge(e1, e2, sc, eW2, cW1, cW2, vecs, hid, with_c):
    ne = e1.shape[0]
    wout = -(-(hid + 16) // 128) * 128 if with_c else hid
    if with_c:
        body = functools.partial(_edge_body_c, hid=hid, pad=wout - hid - 16)
        ins = (e1, e2, sc, eW2, cW1, cW2, vecs)
    else:
        body = functools.partial(_edge_body_nc, hid=hid)
        ins = (e1, e2, sc, eW2, vecs)
    in_specs = [_rows(e1, BE), _rows(e2, BE), _rows(sc, BE)] + [_full(a) for a in ins[3:]]
    outs = jax.ShapeDtypeStruct((ne, wout), jnp.float32)
    out_specs = pl.BlockSpec((BE, wout), lambda i: (i, 0))
    return _tc_call(body, ne // BE, ins, in_specs, outs, out_specs)


def _run_node(h, accs, pos16, nW1, nb1, nW2, nb2, Wa, Wb, ba, hid, fout, hid2):
    wt = _tpad(hid2)
    n_acc = len(accs)
    ins = (h,) + tuple(accs) + (pos16, nW1, nb1, nW2, nb2, Wa, Wb, ba)
    in_specs = ([_rows(h, BN)] + [_rows(a, BN) for a in accs] + [_rows(pos16, BN)]
                + [_full(a) for a in ins[2 + n_acc:]])
    outs = (jax.ShapeDtypeStruct((N, wt), jnp.float32),
            jax.ShapeDtypeStruct((N, wt), jnp.float32),
            jax.ShapeDtypeStruct((N, fout), jnp.float32),
            jax.ShapeDtypeStruct((N, 16), jnp.float32))
    out_specs = (pl.BlockSpec((BN, wt), lambda i: (i, 0)),
                 pl.BlockSpec((BN, wt), lambda i: (i, 0)),
                 pl.BlockSpec((BN, fout), lambda i: (i, 0)),
                 pl.BlockSpec((BN, 16), lambda i: (i, 0)))
    return _tc_call(functools.partial(_node_body, hid=hid, pad=wt - hid2 - 16,
                                      n_acc=n_acc),
                    N // BN, ins, in_specs, outs, out_specs)


def _run_node3(h, accs, nW1, nb1, nW2, nb2, l1W, l1b, l2W, l2b, hid):
    ins = (h,) + tuple(accs) + (nW1, nb1, nW2, nb2, l1W, l1b, l2W, l2b)
    in_specs = ([_rows(h, BN)] + [_rows(a, BN) for a in accs]
                + [_full(a) for a in ins[1 + len(accs):]])
    outs = jax.ShapeDtypeStruct((N, 8), jnp.float32)
    out_specs = pl.BlockSpec((BN, 8), lambda i: (i, 0))
    return _tc_call(functools.partial(_node3_body, hid=hid, n_acc=len(accs)),
                    N // BN, ins, in_specs, outs, out_specs)


_NW = 32
_EB = 128
_NBLK = E // _EB


def _gather(Ta, Tb, row, col):
    Wg = Ta.shape[1]
    ne = row.shape[0]
    nblk = ne // _EB
    mesh = plsc.VectorSubcoreMesh(core_axis_name="c", subcore_axis_name="s")

    @functools.partial(
        pl.kernel, mesh=mesh,
        out_type=(jax.ShapeDtypeStruct((ne, Wg), jnp.float32),
                  jax.ShapeDtypeStruct((ne, Wg), jnp.float32)),
        scratch_types=[pltpu.VMEM((_EB,), jnp.int32),
                       pltpu.VMEM((_EB,), jnp.int32),
                       pltpu.VMEM((_EB, Wg), jnp.float32),
                       pltpu.VMEM((_EB, Wg), jnp.float32),
                       pltpu.SemaphoreType.DMA,
                       pltpu.SemaphoreType.DMA],
    )
    def k(ta_hbm, tb_hbm, row_hbm, col_hbm, e1_hbm, e2_hbm,
          idx1_v, idx2_v, rows1_v, rows2_v, sem1, sem2):
        wid = lax.axis_index("s") * 2 + lax.axis_index("c")

        def body(kk, carry):
            b = kk * _NW + wid

            @pl.when(b < nblk)
            def _():
                base = b * _EB
                pltpu.sync_copy(row_hbm.at[pl.ds(base, _EB)], idx1_v)
                pltpu.sync_copy(col_hbm.at[pl.ds(base, _EB)], idx2_v)
                cp1 = pltpu.async_copy(ta_hbm.at[idx1_v], rows1_v, sem1)
                cp2 = pltpu.async_copy(tb_hbm.at[idx2_v], rows2_v, sem2)
                cp1.wait()
                cp2.wait()
                pltpu.sync_copy(rows1_v, e1_hbm.at[pl.ds(base, _EB)])
                pltpu.sync_copy(rows2_v, e2_hbm.at[pl.ds(base, _EB)])
            return carry

        lax.fori_loop(0, (nblk + _NW - 1) // _NW, body, 0)

    return k(Ta, Tb, row, col)


def _scatter_add(data, row, split_cols):
    Ws = data.shape[1]
    nblk = data.shape[0] // _EB
    zeros = jnp.zeros((640, 128), jnp.float32)
    mesh = plsc.VectorSubcoreMesh(core_axis_name="c", subcore_axis_name="s")
    if split_cols:
        out_t = jax.ShapeDtypeStruct((N, 256), jnp.float32)
    else:
        out_t = jax.ShapeDtypeStruct((2, N, 128), jnp.float32)

    @functools.partial(
        pl.kernel, mesh=mesh,
        out_type=out_t,
        scratch_types=[pltpu.VMEM_SHARED((N, 128), jnp.float32),
                       pltpu.VMEM((_EB,), jnp.int32),
                       pltpu.VMEM((_EB, 128), jnp.float32)],
    )
    def k(data_hbm, row_hbm, z_hbm, out_hbm, acc_sh, idx_v, buf_v):
        cid = lax.axis_index("c")
        sid = lax.axis_index("s")

        def over_rows(fn):
            @pl.when(sid < 15)
            def _():
                fn(pl.multiple_of(sid * 640, 8), 640)

            @pl.when(sid == 15)
            def _():
                fn(9600, 400)

        over_rows(lambda off, sz: pltpu.sync_copy(z_hbm.at[pl.ds(0, sz)],
                                                  acc_sh.at[pl.ds(off, sz)]))
        plsc.subcore_barrier()

        if split_cols:
            def body(kk, carry):
                b = kk * 16 + sid

                @pl.when(b < nblk)
                def _():
                    base = b * _EB
                    pltpu.sync_copy(row_hbm.at[pl.ds(base, _EB)], idx_v)
                    pltpu.sync_copy(
                        data_hbm.at[pl.ds(base, _EB),
                                    pl.ds(pl.multiple_of(cid * 128, 128), 128)],
                        buf_v)
                    pltpu.sync_copy(buf_v, acc_sh.at[idx_v], add=True)
                return carry

            lax.fori_loop(0, (nblk + 15) // 16, body, 0)
            plsc.subcore_barrier()
            over_rows(lambda off, sz: pltpu.sync_copy(
                acc_sh.at[pl.ds(off, sz)],
                out_hbm.at[pl.ds(off, sz),
                           pl.ds(pl.multiple_of(cid * 128, 128), 128)]))
        else:
            def body(kk, carry):
                b = kk * _NW + sid * 2 + cid

                @pl.when(b < nblk)
                def _():
                    base = b * _EB
                    pltpu.sync_copy(row_hbm.at[pl.ds(base, _EB)], idx_v)
                    pltpu.sync_copy(data_hbm.at[pl.ds(base, _EB)], buf_v)
                    pltpu.sync_copy(buf_v, acc_sh.at[idx_v], add=True)
                return carry

            lax.fori_loop(0, (nblk + _NW - 1) // _NW, body, 0)
            plsc.subcore_barrier()
            over_rows(lambda off, sz: pltpu.sync_copy(
                acc_sh.at[pl.ds(off, sz)], out_hbm.at[cid, pl.ds(off, sz)]))

    return k(data, row, zeros)


def kernel(pos, area_point, hks, edge_index, weight, face, vertex2face, di_angles, params):
    row, col = edge_index[0], edge_index[1]
    feat_in = jnp.concatenate([area_point[:, None], hks], axis=1)
    pos16 = jnp.pad(pos, ((0, 0), (0, 13)))
    sc = jnp.concatenate([weight, di_angles[:, None]], axis=1)

    lps = params['layers']
    dims = [(64, 128, 64), (128, 256, 128), (256, 512, 256)]

    def split_e(lp, fin):
        Wa = lp['eW1'][:fin]
        Wb = lp['eW1'][fin:2 * fin]
        return Wa, Wb

    def vecs_of(lp, fin, with_c):
        rows_ = [lp['eW1'][2 * fin], lp['eW1'][2 * fin + 1], lp['eW1'][2 * fin + 2],
                 lp['eb2']]
        rows_ += [lp['cb1']] if with_c else [jnp.zeros_like(lp['eb2'])]
        return jnp.stack(rows_)

    fin0, fout0, hid0 = dims[0]
    Wa0, Wb0 = split_e(lps[0], fin0)
    Ta, Tb, x = _run_pre(feat_in, pos16, params['feat_W'], params['feat_b'][None, :],
                         Wa0, Wb0, lps[0]['eb1'][None, :], hid0)

    parts = 2
    ep = E // parts

    for li, (fin, fout, hid) in enumerate(dims):
        lp = lps[li]
        with_c = li < 2
        accs = []
        for p in range(parts):
            sl = slice(p * ep, (p + 1) * ep)
            e1, e2 = _gather(Ta, Tb, row[sl], col[sl])
            eout = _run_edge(e1, e2, sc[sl], lp['eW2'],
                             lp['cW1'] if with_c else None,
                             lp['cW2'] if with_c else None,
                             vecs_of(lp, fin, with_c), hid, with_c)
            acc = _scatter_add(eout, row[sl], split_cols=eout.shape[1] > 128)
            accs += [acc] if acc.ndim == 2 else [acc[0], acc[1]]
        if with_c:
            fin2, fout2, hid2 = dims[li + 1]
            lp2 = lps[li + 1]
            Wa2, Wb2 = split_e(lp2, fin2)
            Ta, Tb, x, pos16 = _run_node(
                x, accs, pos16, lp['nW1'], lp['nb1'][None, :], lp['nW2'],
                lp['nb2'][None, :], Wa2, Wb2, lp2['eb1'][None, :], hid, fout, hid2)
        else:
            out = _run_node3(x, accs, lp['nW1'], lp['nb1'][None, :], lp['nW2'],
                             lp['nb2'][None, :], params['lin1_W'],
                             params['lin1_b'][None, :], params['lin2_W'],
                             params['lin2_b'][None, :], hid)
    return out
